# Initial kernel scaffold; baseline (speedup 1.0000x reference)
import jax
import jax.numpy as jnp
from jax.experimental import pallas as pl


def kernel(idx, targets, embedding):
    raise NotImplementedError("write your pallas kernel here")


# speedup vs baseline: 1.3418x; 1.3418x over previous
import functools

import jax
import jax.numpy as jnp
from jax import lax
from jax.experimental import pallas as pl
from jax.experimental.pallas import tpu as pltpu
from jax.experimental.pallas import tpu_sc as plsc

V = 1000
N = 51200
NC, NS = 2, 16
NW = NC * NS
BPW = N // NW
CHUNK = 80
NCHUNK = BPW // CHUNK

BLK = 256
G = N // BLK


def _sc_gather(embedding, idx_flat):
    mesh = plsc.VectorSubcoreMesh(core_axis_name="c", subcore_axis_name="s")

    @functools.partial(
        pl.kernel,
        out_type=jax.ShapeDtypeStruct((N, V), jnp.float32),
        mesh=mesh,
        compiler_params=pltpu.CompilerParams(use_tc_tiling_on_sc=False),
        scratch_types=[
            pltpu.VMEM((BPW,), jnp.int32),
            pltpu.VMEM((CHUNK, V), jnp.float32),
            pltpu.SemaphoreType.DMA,
        ],
    )
    def k(table_hbm, idx_hbm, out_hbm, idx_v, rows_v, sem):
        wid = lax.axis_index("s") * NC + lax.axis_index("c")
        base = wid * BPW
        pltpu.sync_copy(idx_hbm.at[pl.ds(base, BPW)], idx_v)

        @pl.loop(0, NCHUNK)
        def _(c):
            off = c * CHUNK
            pltpu.async_copy(
                table_hbm.at[idx_v.at[pl.ds(off, CHUNK)]], rows_v, sem
            ).wait()
            pltpu.sync_copy(rows_v, out_hbm.at[pl.ds(base + off, CHUNK)])

    return k(embedding, idx_flat)


def _tc_ce_sum(logits, targets_col):
    def body(t_ref, x_ref, loss_ref):
        i = pl.program_id(0)
        rows = x_ref[...]
        m = jnp.max(rows, axis=1, keepdims=True)
        s = jnp.sum(jnp.exp(rows - m), axis=1, keepdims=True)
        lse = m + jnp.log(s)
        lane = lax.broadcasted_iota(jnp.int32, (BLK, V), 1)
        val = jnp.max(
            jnp.where(lane == t_ref[...], rows, jnp.float32(-1e30)),
            axis=1, keepdims=True,
        )
        part = jnp.sum(lse - val)

        @pl.when(i == 0)
        def _():
            loss_ref[0, 0] = 0.0

        loss_ref[0, 0] += part

    return pl.pallas_call(
        body,
        grid=(G,),
        in_specs=[
            pl.BlockSpec((BLK, 1), lambda i: (i, 0)),
            pl.BlockSpec((BLK, V), lambda i: (i, 0)),
        ],
        out_specs=pl.BlockSpec(
            block_shape=(1, 1), index_map=lambda i: (0, 0),
            memory_space=pltpu.SMEM,
        ),
        out_shape=jax.ShapeDtypeStruct((1, 1), jnp.float32),
    )(targets_col, logits)


def kernel(idx, targets, embedding):
    idx_flat = idx.reshape(-1)
    logits = _sc_gather(embedding, idx_flat)
    loss_sum = _tc_ce_sum(logits, targets.reshape(-1, 1))
    return logits, loss_sum[0, 0] / jnp.float32(N)


# speedup vs baseline: 1.9678x; 1.4665x over previous
import functools

import jax
import jax.numpy as jnp
from jax import lax
from jax.experimental import pallas as pl
from jax.experimental.pallas import tpu as pltpu
from jax.experimental.pallas import tpu_sc as plsc

V = 1000
VP = 1024
N = 51200
NC, NS = 2, 16
NW = NC * NS
BPW = N // NW
CHUNK = 80
NCHUNK = BPW // CHUNK

BLK = 256
G = N // BLK


def _sc_gather(embedding, idx_flat):
    mesh = plsc.VectorSubcoreMesh(core_axis_name="c", subcore_axis_name="s")

    @functools.partial(
        pl.kernel,
        out_type=jax.ShapeDtypeStruct((N, VP), jnp.float32),
        mesh=mesh,
        scratch_types=[
            pltpu.VMEM((BPW,), jnp.int32),
            pltpu.VMEM((CHUNK, VP), jnp.float32),
            pltpu.SemaphoreType.DMA,
        ],
    )
    def k(table_hbm, idx_hbm, out_hbm, idx_v, rows_v, sem):
        wid = lax.axis_index("s") * NC + lax.axis_index("c")
        base = wid * BPW
        pltpu.sync_copy(idx_hbm.at[pl.ds(base, BPW)], idx_v)

        @pl.loop(0, NCHUNK)
        def _(c):
            off = c * CHUNK
            pltpu.async_copy(
                table_hbm.at[idx_v.at[pl.ds(off, CHUNK)]], rows_v, sem
            ).wait()
            pltpu.sync_copy(rows_v, out_hbm.at[pl.ds(base + off, CHUNK)])

    return k(embedding, idx_flat)


def _tc_ce_sum(logits, targets_col):
    def body(t_ref, x_ref, loss_ref):
        i = pl.program_id(0)
        rows = x_ref[...]
        m = jnp.max(rows, axis=1, keepdims=True)
        s = jnp.sum(jnp.exp(rows - m), axis=1, keepdims=True)
        lse = m + jnp.log(s)
        lane = lax.broadcasted_iota(jnp.int32, (BLK, VP), 1)
        val = jnp.max(
            jnp.where(lane == t_ref[...], rows, jnp.float32(-1e30)),
            axis=1, keepdims=True,
        )
        part = jnp.sum(lse - val)

        @pl.when(i == 0)
        def _():
            loss_ref[0, 0] = 0.0

        loss_ref[0, 0] += part

    return pl.pallas_call(
        body,
        grid=(G,),
        in_specs=[
            pl.BlockSpec((BLK, 1), lambda i: (i, 0)),
            pl.BlockSpec((BLK, VP), lambda i: (i, 0)),
        ],
        out_specs=pl.BlockSpec(
            block_shape=(1, 1), index_map=lambda i: (0, 0),
            memory_space=pltpu.SMEM,
        ),
        out_shape=jax.ShapeDtypeStruct((1, 1), jnp.float32),
    )(targets_col, logits)


def kernel(idx, targets, embedding):
    idx_flat = idx.reshape(-1)
    emb_p = jnp.pad(embedding, ((0, 0), (0, VP - V)),
                    constant_values=jnp.float32(-1e30))
    out_p = _sc_gather(emb_p, idx_flat)
    loss_sum = _tc_ce_sum(out_p, targets.reshape(-1, 1))
    return out_p[:, :V], loss_sum[0, 0] / jnp.float32(N)
